# Initial kernel scaffold; baseline (speedup 1.0000x reference)
import jax
import jax.numpy as jnp
from jax.experimental import pallas as pl


def kernel(x, edge_index, batch, W1, a_src1, a_dst1, b1, W2, a_src2, a_dst2, b2, Wf1, bf1, Wf2, bf2):
    raise NotImplementedError("write your pallas kernel here")


# speedup vs baseline: 49.3933x; 49.3933x over previous
import functools

import jax
import jax.numpy as jnp
from jax import lax
from jax.experimental import pallas as pl
from jax.experimental.pallas import tpu as pltpu
from jax.experimental.pallas import tpu_sc as plsc

N = 10000
E = 160000
F_IN = 256
HEADS = 8
OUT = 8
F = HEADS * OUT
G = 64
NCLS = 2

N_PAD = 10240
NRANGES = 4
RANGE = N_PAD // NRANGES
NTILES = 32
EPT = E // NTILES
CHUNK = 512
CAP = 5120
BLK = 512
NBLK = N_PAD // BLK

_sc_mesh = plsc.VectorSubcoreMesh(core_axis_name="c", subcore_axis_name="s",
                                  num_cores=2, num_subcores=16)
_sc_params = pltpu.CompilerParams(needs_layout_passes=False)


@functools.partial(
    pl.kernel,
    out_type=(
        jax.ShapeDtypeStruct((NRANGES, NTILES, CAP), jnp.int32),
        jax.ShapeDtypeStruct((NRANGES, NTILES, CAP), jnp.int32),
        jax.ShapeDtypeStruct((NTILES, 16), jnp.int32),
    ),
    mesh=_sc_mesh,
    compiler_params=_sc_params,
    scratch_types=[
        pltpu.VMEM((EPT + 16,), jnp.int32),
        pltpu.VMEM((EPT + 16,), jnp.int32),
        pltpu.VMEM((NRANGES, CAP), jnp.int32),
        pltpu.VMEM((NRANGES, CAP), jnp.int32),
        pltpu.VMEM((16,), jnp.int32),
    ],
)
def _bucket_kernel(src_hbm, dst_hbm, bsrc_hbm, bdst_hbm, cnt_hbm,
                   src_loc, dst_loc, bsrc_loc, bdst_loc, cnt_loc):
    cid = lax.axis_index("c")
    sid = lax.axis_index("s")
    wid = sid * 2 + cid
    base = wid * EPT
    pltpu.sync_copy(src_hbm.at[pl.ds(base, EPT)], src_loc.at[pl.ds(0, EPT)])
    pltpu.sync_copy(dst_hbm.at[pl.ds(base, EPT)], dst_loc.at[pl.ds(0, EPT)])
    lane = lax.iota(jnp.int32, 16)
    ngroups = (EPT + 15) // 16

    def body(i, offs):
        src_v = src_loc[pl.ds(i * 16, 16)]
        dst_v = dst_loc[pl.ds(i * 16, 16)]
        valid = (i * 16 + lane) < EPT
        r_v = ((dst_v >= RANGE).astype(jnp.int32)
               + (dst_v >= 2 * RANGE).astype(jnp.int32)
               + (dst_v >= 3 * RANGE).astype(jnp.int32))
        pos = jnp.zeros((16,), jnp.int32)
        new_offs = []
        for r in range(NRANGES):
            m = valid & (r_v == r)
            mi = m.astype(jnp.int32)
            rank = plsc.cumsum(mi) - 1
            pos = jnp.where(m, offs[r] + rank, pos)
            new_offs.append(offs[r] + jnp.sum(mi))
        plsc.store_scatter(bsrc_loc, [r_v, pos], src_v, mask=valid)
        plsc.store_scatter(bdst_loc, [r_v, pos], dst_v, mask=valid)
        return tuple(new_offs)

    offs = lax.fori_loop(0, ngroups, body, (0, 0, 0, 0))

    cnt_v = jnp.zeros((16,), jnp.int32)
    for r in range(NRANGES):
        cnt_v = jnp.where(lane == r, offs[r], cnt_v)
    cnt_loc[...] = cnt_v
    pltpu.sync_copy(cnt_loc, cnt_hbm.at[wid])
    for r in range(NRANGES):
        pltpu.sync_copy(bsrc_loc.at[r], bsrc_hbm.at[r, wid])
        pltpu.sync_copy(bdst_loc.at[r], bdst_hbm.at[r, wid])


@functools.partial(
    pl.kernel,
    out_type=(
        jax.ShapeDtypeStruct((F, N_PAD), jnp.float32),
        jax.ShapeDtypeStruct((HEADS, N_PAD), jnp.float32),
    ),
    mesh=_sc_mesh,
    compiler_params=_sc_params,
    scratch_types=[
        pltpu.VMEM((OUT, N_PAD), jnp.float32),
        pltpu.VMEM((N_PAD,), jnp.float32),
        pltpu.VMEM((RANGE,), jnp.float32),
        pltpu.VMEM((OUT, RANGE), jnp.float32),
        pltpu.VMEM((RANGE,), jnp.float32),
        pltpu.VMEM((NTILES, 16), jnp.int32),
        pltpu.VMEM((CHUNK,), jnp.int32),
        pltpu.VMEM((CHUNK,), jnp.int32),
    ],
)
def _gat_edges_kernel(bsrc_hbm, bdst_hbm, cnt_hbm, ht_hbm, asrc_hbm, adst_hbm,
                      agg_hbm, den_hbm,
                      h_loc, asrc_loc, adst_loc, agg_loc, den_loc, cnt_loc,
                      src_buf, dst_buf):
    cid = lax.axis_index("c")
    sid = lax.axis_index("s")
    wid = sid * 2 + cid
    hd = wid // NRANGES
    r = wid % NRANGES
    r0 = r * RANGE
    lane = lax.iota(jnp.int32, 16)

    pltpu.sync_copy(ht_hbm.at[pl.ds(hd * OUT, OUT)], h_loc)
    pltpu.sync_copy(asrc_hbm.at[hd], asrc_loc)
    pltpu.sync_copy(adst_hbm.at[hd, pl.ds(r0, RANGE)], adst_loc)
    pltpu.sync_copy(cnt_hbm, cnt_loc)

    zf = jnp.zeros((16,), jnp.float32)
    for o in range(OUT):
        def zbody(i, _, o=o):
            agg_loc[o, pl.ds(i * 16, 16)] = zf
            return 0
        lax.fori_loop(0, RANGE // 16, zbody, 0)

    def zdbody(i, _):
        den_loc[pl.ds(i * 16, 16)] = zf
        return 0
    lax.fori_loop(0, RANGE // 16, zdbody, 0)

    for t in range(NTILES):
        cnt_row = cnt_loc[t]
        cnt = jnp.max(jnp.where(lane == r, cnt_row, 0))
        nchunks = (cnt + CHUNK - 1) // CHUNK

        def chunk_body(c, _, t=t):
            pltpu.sync_copy(bsrc_hbm.at[r, t, pl.ds(c * CHUNK, CHUNK)], src_buf)
            pltpu.sync_copy(bdst_hbm.at[r, t, pl.ds(c * CHUNK, CHUNK)], dst_buf)

            def group_body(j, _, c=c):
                src_v = src_buf[pl.ds(j * 16, 16)]
                dst_v = dst_buf[pl.ds(j * 16, 16)]
                m = (c * CHUNK + j * 16 + lane) < cnt
                src_c = jnp.clip(src_v, 0, N - 1)
                drel = jnp.clip(dst_v - r0, 0, RANGE - 1)
                a_s = plsc.load_gather(asrc_loc, [src_c], mask=m)
                a_d = plsc.load_gather(adst_loc, [drel], mask=m)
                e = a_s + a_d
                e = jnp.maximum(e, 0.2 * e)
                ex = jnp.exp(e)
                ex = jnp.where(m, ex, 0.0)
                plsc.addupdate_scatter(den_loc, [drel], ex, mask=m)
                for o in range(OUT):
                    o_v = jnp.full((16,), o, jnp.int32)
                    hv = plsc.load_gather(h_loc, [o_v, src_c], mask=m)
                    plsc.addupdate_scatter(agg_loc, [o_v, drel], hv * ex,
                                           mask=m)
                return 0

            lax.fori_loop(0, CHUNK // 16, group_body, 0)
            return 0

        lax.fori_loop(0, nchunks, chunk_body, 0)

    for o in range(OUT):
        pltpu.sync_copy(agg_loc.at[o],
                        agg_hbm.at[hd * OUT + o, pl.ds(r0, RANGE)])
    pltpu.sync_copy(den_loc, den_hbm.at[hd, pl.ds(r0, RANGE)])


def _tc1_body(x_ref, w_ref, asrc_ref, adst_ref, ht_ref, at_s_ref, at_d_ref):
    h = jnp.dot(x_ref[...], w_ref[...], preferred_element_type=jnp.float32)
    ht_ref[...] = h.T
    hr = h.reshape(BLK, HEADS, OUT)
    at_s_ref[...] = jnp.sum(hr * asrc_ref[...][None], axis=-1).T
    at_d_ref[...] = jnp.sum(hr * adst_ref[...][None], axis=-1).T


def _tc1(x_pad, W1, a_src1, a_dst1):
    return pl.pallas_call(
        _tc1_body,
        grid=(NBLK,),
        in_specs=[
            pl.BlockSpec((BLK, F_IN), lambda i: (i, 0)),
            pl.BlockSpec((F_IN, F), lambda i: (0, 0)),
            pl.BlockSpec((HEADS, OUT), lambda i: (0, 0)),
            pl.BlockSpec((HEADS, OUT), lambda i: (0, 0)),
        ],
        out_specs=[
            pl.BlockSpec((F, BLK), lambda i: (0, i)),
            pl.BlockSpec((HEADS, BLK), lambda i: (0, i)),
            pl.BlockSpec((HEADS, BLK), lambda i: (0, i)),
        ],
        out_shape=[
            jax.ShapeDtypeStruct((F, N_PAD), jnp.float32),
            jax.ShapeDtypeStruct((HEADS, N_PAD), jnp.float32),
            jax.ShapeDtypeStruct((HEADS, N_PAD), jnp.float32),
        ],
    )(x_pad, W1, a_src1, a_dst1)


def _tc2_body(agg_ref, den_ref, b1_ref, w2_ref, asrc_ref, adst_ref,
              ht_ref, at_s_ref, at_d_ref):
    agg = agg_ref[...]
    den = den_ref[...]
    den64 = jnp.broadcast_to(den[:, None, :], (HEADS, OUT, BLK)).reshape(F, BLK)
    out1 = agg / (den64 + 1e-16) + b1_ref[...][:, None]
    h2in = jnp.where(out1 > 0, out1, jnp.exp(jnp.minimum(out1, 0.0)) - 1.0)
    h2 = lax.dot_general(h2in, w2_ref[...], (((0,), (0,)), ((), ())),
                         preferred_element_type=jnp.float32)
    ht_ref[...] = h2.T
    hr = h2.reshape(BLK, HEADS, OUT)
    at_s_ref[...] = jnp.sum(hr * asrc_ref[...][None], axis=-1).T
    at_d_ref[...] = jnp.sum(hr * adst_ref[...][None], axis=-1).T


def _tc2(agg1, den1, b1, W2, a_src2, a_dst2):
    return pl.pallas_call(
        _tc2_body,
        grid=(NBLK,),
        in_specs=[
            pl.BlockSpec((F, BLK), lambda i: (0, i)),
            pl.BlockSpec((HEADS, BLK), lambda i: (0, i)),
            pl.BlockSpec((F,), lambda i: (0,)),
            pl.BlockSpec((F, F), lambda i: (0, 0)),
            pl.BlockSpec((HEADS, OUT), lambda i: (0, 0)),
            pl.BlockSpec((HEADS, OUT), lambda i: (0, 0)),
        ],
        out_specs=[
            pl.BlockSpec((F, BLK), lambda i: (0, i)),
            pl.BlockSpec((HEADS, BLK), lambda i: (0, i)),
            pl.BlockSpec((HEADS, BLK), lambda i: (0, i)),
        ],
        out_shape=[
            jax.ShapeDtypeStruct((F, N_PAD), jnp.float32),
            jax.ShapeDtypeStruct((HEADS, N_PAD), jnp.float32),
            jax.ShapeDtypeStruct((HEADS, N_PAD), jnp.float32),
        ],
    )(agg1, den1, b1, W2, a_src2, a_dst2)


def _tc3_body(agg_ref, den_ref, b2_ref, batch_ref, wf1_ref, bf1_ref,
              wf2_ref, bf2_ref, g_ref, out_ref):
    i = pl.program_id(0)
    agg = agg_ref[...]
    den = den_ref[...]
    den64 = jnp.broadcast_to(den[:, None, :], (HEADS, OUT, BLK)).reshape(F, BLK)
    out2 = agg / (den64 + 1e-16) + b2_ref[...][:, None]
    b = batch_ref[0, 0, :]
    onehot = (b[None, :] == lax.broadcasted_iota(jnp.int32, (G, BLK), 0))
    onehot = onehot.astype(jnp.float32)
    part = lax.dot_general(onehot, out2, (((1,), (1,)), ((), ())),
                           preferred_element_type=jnp.float32)

    @pl.when(i == 0)
    def _():
        g_ref[...] = jnp.zeros_like(g_ref)

    g_ref[...] += part

    @pl.when(i == NBLK - 1)
    def _():
        g = jnp.maximum(g_ref[...] @ wf1_ref[...] + bf1_ref[...][None, :], 0.0)
        logits = g @ wf2_ref[...] + bf2_ref[...][None, :]
        m = jnp.max(logits, axis=-1, keepdims=True)
        lse = m + jnp.log(jnp.sum(jnp.exp(logits - m), axis=-1, keepdims=True))
        out_ref[...] = logits - lse


def _tc3(agg2, den2, b2, batch_pad, Wf1, bf1, Wf2, bf2):
    batch3 = batch_pad.reshape(NBLK, 1, BLK)
    return pl.pallas_call(
        _tc3_body,
        grid=(NBLK,),
        in_specs=[
            pl.BlockSpec((F, BLK), lambda i: (0, i)),
            pl.BlockSpec((HEADS, BLK), lambda i: (0, i)),
            pl.BlockSpec((F,), lambda i: (0,)),
            pl.BlockSpec((1, 1, BLK), lambda i: (i, 0, 0)),
            pl.BlockSpec((F, 32), lambda i: (0, 0)),
            pl.BlockSpec((32,), lambda i: (0,)),
            pl.BlockSpec((32, NCLS), lambda i: (0, 0)),
            pl.BlockSpec((NCLS,), lambda i: (0,)),
        ],
        out_specs=[
            pl.BlockSpec((G, F), lambda i: (0, 0)),
            pl.BlockSpec((G, NCLS), lambda i: (0, 0)),
        ],
        out_shape=[
            jax.ShapeDtypeStruct((G, F), jnp.float32),
            jax.ShapeDtypeStruct((G, NCLS), jnp.float32),
        ],
    )(agg2, den2, b2, batch3, Wf1, bf1, Wf2, bf2)


def kernel(x, edge_index, batch, W1, a_src1, a_dst1, b1, W2, a_src2, a_dst2,
           b2, Wf1, bf1, Wf2, bf2):
    src = edge_index[0].astype(jnp.int32)
    dst = edge_index[1].astype(jnp.int32)
    x_pad = jnp.pad(x, ((0, N_PAD - N), (0, 0)))
    batch_pad = jnp.pad(batch.astype(jnp.int32), (0, N_PAD - N),
                        constant_values=127)

    bsrc, bdst, cnts = _bucket_kernel(src, dst)

    ht1, asrc1_t, adst1_t = _tc1(x_pad, W1, a_src1, a_dst1)
    agg1, den1 = _gat_edges_kernel(bsrc, bdst, cnts, ht1, asrc1_t, adst1_t)

    ht2, asrc2_t, adst2_t = _tc2(agg1, den1, b1, W2, a_src2, a_dst2)
    agg2, den2 = _gat_edges_kernel(bsrc, bdst, cnts, ht2, asrc2_t, adst2_t)

    _, logp = _tc3(agg2, den2, b2, batch_pad, Wf1, bf1, Wf2, bf2)
    return logp
